# baseline (device time: 37841 ns/iter reference)
import jax
import jax.numpy as jnp
from jax import lax
from jax.experimental import pallas as pl
from jax.experimental.pallas import tpu as pltpu

K = 16
EXTRACT_PER_PASS = 8
NEG = float("-inf")
N_Z = 4


def _lane_idx(shape):
    return lax.broadcasted_iota(jnp.int32, shape, 1)


def _rev16(v):
    idx = _lane_idx(v.shape)
    for d in (8, 4, 2, 1):
        v = jnp.where((idx & d) == 0, pltpu.roll(v, K - d, 1), pltpu.roll(v, d, 1))
    return v


def _bitonic_desc(v):
    idx = _lane_idx(v.shape)
    for d in (8, 4, 2, 1):
        mask = (idx & d) == 0
        partner = jnp.where(mask, pltpu.roll(v, K - d, 1), pltpu.roll(v, d, 1))
        v = jnp.where(mask, jnp.maximum(v, partner), jnp.minimum(v, partner))
    return v


def _merge_desc(a, b):
    return _bitonic_desc(jnp.maximum(a, _rev16(b)))


def kernel(x):
    m, n = x.shape

    def body(x_ref, o_ref, work_ref, cand_ref, recv_ref, send_sems, recv_sems):
        my_x = lax.axis_index("x")
        my_y = lax.axis_index("y")
        my_z = lax.axis_index("z")

        barrier = pltpu.get_barrier_semaphore()
        for t in range(1, N_Z):
            pl.semaphore_signal(
                barrier,
                inc=1,
                device_id=(my_x, my_y, my_z ^ t),
                device_id_type=pl.DeviceIdType.MESH,
            )
        pl.semaphore_wait(barrier, N_Z - 1)

        v = x_ref[:, :]
        for p in range(K // EXTRACT_PER_PASS):
            if p > 0:
                v = work_ref[:, :]
            for e in range(EXTRACT_PER_PASS):
                j = p * EXTRACT_PER_PASS + e
                mx = jnp.max(v, axis=1, keepdims=True)
                cand_ref[:, j : j + 1] = mx
                if j < K - 1:
                    v = jnp.where(v == mx, NEG, v)
            if p < K // EXTRACT_PER_PASS - 1:
                work_ref[:, :] = v

        rdmas = []
        for t in range(1, N_Z):
            rdma = pltpu.make_async_remote_copy(
                src_ref=cand_ref,
                dst_ref=recv_ref.at[t - 1],
                send_sem=send_sems.at[t - 1],
                recv_sem=recv_sems.at[t - 1],
                device_id=(my_x, my_y, my_z ^ t),
                device_id_type=pl.DeviceIdType.MESH,
            )
            rdma.start()
            rdmas.append(rdma)
        for rdma in rdmas:
            rdma.wait()

        m01 = _merge_desc(cand_ref[:, :], recv_ref[0, :, :])
        m23 = _merge_desc(recv_ref[1, :, :], recv_ref[2, :, :])
        o_ref[:, :] = _merge_desc(m01, m23)

    return pl.pallas_call(
        body,
        out_shape=jax.ShapeDtypeStruct((m, K), jnp.float32),
        in_specs=[pl.BlockSpec(memory_space=pltpu.VMEM)],
        out_specs=pl.BlockSpec(memory_space=pltpu.VMEM),
        scratch_shapes=[
            pltpu.VMEM((m, n), jnp.float32),
            pltpu.VMEM((m, K), jnp.float32),
            pltpu.VMEM((N_Z - 1, m, K), jnp.float32),
            pltpu.SemaphoreType.DMA((N_Z - 1,)),
            pltpu.SemaphoreType.DMA((N_Z - 1,)),
        ],
        compiler_params=pltpu.CompilerParams(collective_id=0),
    )(x)


# device time: 24768 ns/iter; 1.5278x vs baseline; 1.5278x over previous
import jax
import jax.numpy as jnp
from jax import lax
from jax.experimental import pallas as pl
from jax.experimental.pallas import tpu as pltpu

K = 16
EXTRACT_PER_PASS = 8
NEG = float("-inf")
N_Z = 4
N_XY = 4


def _lane_idx(shape):
    return lax.broadcasted_iota(jnp.int32, shape, 1)


def _rev16(v):
    idx = _lane_idx(v.shape)
    for d in (8, 4, 2, 1):
        v = jnp.where((idx & d) == 0, pltpu.roll(v, K - d, 1), pltpu.roll(v, d, 1))
    return v


def _bitonic_desc(v):
    idx = _lane_idx(v.shape)
    for d in (8, 4, 2, 1):
        mask = (idx & d) == 0
        partner = jnp.where(mask, pltpu.roll(v, K - d, 1), pltpu.roll(v, d, 1))
        v = jnp.where(mask, jnp.maximum(v, partner), jnp.minimum(v, partner))
    return v


def _merge_desc(a, b):
    return _bitonic_desc(jnp.maximum(a, _rev16(b)))


def kernel(x):
    m, n = x.shape
    mb = m // N_XY

    def body(
        x_ref,
        o_ref,
        work_ref,
        cand_ref,
        zrecv_ref,
        z_send_sems,
        z_recv_sems,
        xy_send_sems,
        xy_recv_sems,
    ):
        my_x = lax.axis_index("x")
        my_y = lax.axis_index("y")
        my_z = lax.axis_index("z")
        rb = my_x * 2 + my_y
        row0 = rb * mb

        peers = [
            (my_x, my_y, my_z ^ 1),
            (my_x, my_y, my_z ^ 2),
            (my_x, my_y ^ 1, my_z),
            (my_x ^ 1, my_y, my_z),
            (my_x ^ 1, my_y ^ 1, my_z),
        ]

        barrier = pltpu.get_barrier_semaphore()
        for p in peers:
            pl.semaphore_signal(
                barrier, inc=1, device_id=p, device_id_type=pl.DeviceIdType.MESH
            )
        pl.semaphore_wait(barrier, len(peers))

        v = x_ref[pl.ds(row0, mb), :]
        for p in range(K // EXTRACT_PER_PASS):
            if p > 0:
                v = work_ref[:, :]
            for e in range(EXTRACT_PER_PASS):
                j = p * EXTRACT_PER_PASS + e
                mx = jnp.max(v, axis=1, keepdims=True)
                cand_ref[:, j : j + 1] = mx
                if j < K - 1:
                    v = jnp.where(v == mx, NEG, v)
            if p < K // EXTRACT_PER_PASS - 1:
                work_ref[:, :] = v

        for r in range(2):
            rdma = pltpu.make_async_remote_copy(
                src_ref=cand_ref,
                dst_ref=zrecv_ref.at[r],
                send_sem=z_send_sems.at[r],
                recv_sem=z_recv_sems.at[r],
                device_id=(my_x, my_y, my_z ^ (1 << r)),
                device_id_type=pl.DeviceIdType.MESH,
            )
            rdma.start()
            rdma.wait()
            cand_ref[:, :] = _merge_desc(cand_ref[:, :], zrecv_ref[r, :, :])

        o_ref[pl.ds(row0, mb), :] = cand_ref[:, :]
        rdmas = []
        for t, (dx, dy) in enumerate(((0, 1), (1, 0), (1, 1))):
            rdma = pltpu.make_async_remote_copy(
                src_ref=cand_ref,
                dst_ref=o_ref.at[pl.ds(row0, mb), :],
                send_sem=xy_send_sems.at[t],
                recv_sem=xy_recv_sems.at[t],
                device_id=(my_x ^ dx, my_y ^ dy, my_z),
                device_id_type=pl.DeviceIdType.MESH,
            )
            rdma.start()
            rdmas.append(rdma)
        for rdma in rdmas:
            rdma.wait()

    return pl.pallas_call(
        body,
        out_shape=jax.ShapeDtypeStruct((m, K), jnp.float32),
        in_specs=[pl.BlockSpec(memory_space=pltpu.VMEM)],
        out_specs=pl.BlockSpec(memory_space=pltpu.VMEM),
        scratch_shapes=[
            pltpu.VMEM((mb, n), jnp.float32),
            pltpu.VMEM((mb, K), jnp.float32),
            pltpu.VMEM((2, mb, K), jnp.float32),
            pltpu.SemaphoreType.DMA((2,)),
            pltpu.SemaphoreType.DMA((2,)),
            pltpu.SemaphoreType.DMA((3,)),
            pltpu.SemaphoreType.DMA((3,)),
        ],
        compiler_params=pltpu.CompilerParams(collective_id=0),
    )(x)


# device time: 21438 ns/iter; 1.7651x vs baseline; 1.1553x over previous
import jax
import jax.numpy as jnp
from jax import lax
from jax.experimental import pallas as pl
from jax.experimental.pallas import tpu as pltpu

K = 16
N_STRIPS = 4
NEG = float("-inf")
N_Z = 4
N_XY = 4


def kernel(x):
    m, n = x.shape
    mb = m // N_XY
    ns = n // N_STRIPS

    def body(
        x_ref,
        o_ref,
        lcand_ref,
        cand_ref,
        zrecv_ref,
        z_send_sems,
        z_recv_sems,
        xy_send_sems,
        xy_recv_sems,
    ):
        my_x = lax.axis_index("x")
        my_y = lax.axis_index("y")
        my_z = lax.axis_index("z")
        rb = my_x * 2 + my_y
        row0 = rb * mb

        peers = [(my_x, my_y, my_z ^ t) for t in range(1, N_Z)] + [
            (my_x, my_y ^ 1, my_z),
            (my_x ^ 1, my_y, my_z),
            (my_x ^ 1, my_y ^ 1, my_z),
        ]

        barrier = pltpu.get_barrier_semaphore()
        for p in peers:
            pl.semaphore_signal(
                barrier, inc=1, device_id=p, device_id_type=pl.DeviceIdType.MESH
            )
        pl.semaphore_wait(barrier, len(peers))

        for s in range(N_STRIPS):
            v = x_ref[pl.ds(row0, mb), s * ns : (s + 1) * ns]
            for j in range(K):
                mx = jnp.max(v, axis=1, keepdims=True)
                c = s * K + j
                lcand_ref[:, c : c + 1] = mx
                if j < K - 1:
                    v = jnp.where(v == mx, NEG, v)

        rdmas = []
        for t in range(1, N_Z):
            rdma = pltpu.make_async_remote_copy(
                src_ref=lcand_ref,
                dst_ref=zrecv_ref.at[t - 1],
                send_sem=z_send_sems.at[t - 1],
                recv_sem=z_recv_sems.at[t - 1],
                device_id=(my_x, my_y, my_z ^ t),
                device_id_type=pl.DeviceIdType.MESH,
            )
            rdma.start()
            rdmas.append(rdma)
        for rdma in rdmas:
            rdma.wait()

        v = jnp.concatenate(
            [lcand_ref[:, :]]
            + [zrecv_ref[t, :, :] for t in range(N_Z - 1)],
            axis=1,
        )
        for j in range(K):
            mx = jnp.max(v, axis=1, keepdims=True)
            cand_ref[:, j : j + 1] = mx
            if j < K - 1:
                v = jnp.where(v == mx, NEG, v)

        o_ref[pl.ds(row0, mb), :] = cand_ref[:, :]
        rdmas = []
        for t, (dx, dy) in enumerate(((0, 1), (1, 0), (1, 1))):
            rdma = pltpu.make_async_remote_copy(
                src_ref=cand_ref,
                dst_ref=o_ref.at[pl.ds(row0, mb), :],
                send_sem=xy_send_sems.at[t],
                recv_sem=xy_recv_sems.at[t],
                device_id=(my_x ^ dx, my_y ^ dy, my_z),
                device_id_type=pl.DeviceIdType.MESH,
            )
            rdma.start()
            rdmas.append(rdma)
        for rdma in rdmas:
            rdma.wait()

    return pl.pallas_call(
        body,
        out_shape=jax.ShapeDtypeStruct((m, K), jnp.float32),
        in_specs=[pl.BlockSpec(memory_space=pltpu.VMEM)],
        out_specs=pl.BlockSpec(memory_space=pltpu.VMEM),
        scratch_shapes=[
            pltpu.VMEM((mb, N_STRIPS * K), jnp.float32),
            pltpu.VMEM((mb, K), jnp.float32),
            pltpu.VMEM((N_Z - 1, mb, N_STRIPS * K), jnp.float32),
            pltpu.SemaphoreType.DMA((N_Z - 1,)),
            pltpu.SemaphoreType.DMA((N_Z - 1,)),
            pltpu.SemaphoreType.DMA((3,)),
            pltpu.SemaphoreType.DMA((3,)),
        ],
        compiler_params=pltpu.CompilerParams(collective_id=0),
    )(x)
